# Initial kernel scaffold; baseline (speedup 1.0000x reference)
import jax
import jax.numpy as jnp
from jax.experimental import pallas as pl


def kernel(inputs, targets):
    raise NotImplementedError("write your pallas kernel here")


# speedup vs baseline: 8.0596x; 8.0596x over previous
import jax
import jax.numpy as jnp
from jax.experimental import pallas as pl
from jax.experimental.pallas import tpu as pltpu

_B, _C, _K = 4096, 10000, 3
_BB = 128


def _loss_body(x_ref, t_ref, out_ref):
    i = pl.program_id(0)
    v = x_ref[...]
    x = jax.nn.log_sigmoid(v)
    recip = jnp.sum(1.0 / x, axis=1)
    tb = t_ref[...]
    cols = jax.lax.broadcasted_iota(jnp.int32, (_BB, _C), 1)
    mask = (cols == tb[:, 0:1]) | (cols == tb[:, 1:2]) | (cols == tb[:, 2:3])
    ts = jnp.maximum(jnp.sum(jnp.where(mask, x, 0.0), axis=1), 1e-6)
    partial = jnp.sum(ts * recip).reshape(1, 1)

    @pl.when(i == 0)
    def _init():
        out_ref[...] = jnp.zeros((1, 1), jnp.float32)

    out_ref[...] += partial


def kernel(inputs, targets):
    x2d = inputs[..., 0]
    grid = _B // _BB
    acc = pl.pallas_call(
        _loss_body,
        grid=(grid,),
        in_specs=[
            pl.BlockSpec((_BB, _C), lambda i: (i, 0)),
            pl.BlockSpec((_BB, _K), lambda i: (i, 0)),
        ],
        out_specs=pl.BlockSpec((1, 1), lambda i: (0, 0)),
        out_shape=jax.ShapeDtypeStruct((1, 1), jnp.float32),
        compiler_params=pltpu.CompilerParams(
            dimension_semantics=("arbitrary",),
        ),
    )(x2d, targets)
    return -acc[0, 0] / _B
